# Initial kernel scaffold; baseline (speedup 1.0000x reference)
import jax
import jax.numpy as jnp
from jax.experimental import pallas as pl


def kernel(x, edge_index, edge_attr, node_w1, node_b1, node_w2, node_b2, edge_w1, edge_b1, edge_w2, edge_b2, msg_w1, msg_b1, msg_w2, msg_b2, upd_w1, upd_b1, upd_w2, upd_b2, ln_g, ln_b):
    raise NotImplementedError("write your pallas kernel here")


# speedup vs baseline: 1.2827x; 1.2827x over previous
import functools

import jax
import jax.numpy as jnp
from jax import lax
from jax.experimental import pallas as pl
from jax.experimental.pallas import tpu as pltpu
from jax.experimental.pallas import tpu_sc as plsc

N = 10000
E = 320000
D = 128
DE = 16
H = 128

NC = 2
NS = 16
CHUNK = 80
EP_TILE = E // (NC * NS)
NCHUNK = EP_TILE // CHUNK
AW = H + 16
NPAD = 10240


def _f32(x):
    return x.astype(jnp.float32)


def _node_body(x_ref, nw1_ref, nb1_ref, nw2_ref, nb2_ref, wi_ref, wj_ref,
               a_ref, b_ref):
    h1 = jax.nn.silu(
        jnp.dot(x_ref[...], nw1_ref[...], preferred_element_type=jnp.float32)
        + nb1_ref[...])
    h = jnp.dot(h1, nw2_ref[...], preferred_element_type=jnp.float32) + nb2_ref[...]
    a_ref[...] = jnp.dot(h, wi_ref[...], preferred_element_type=jnp.float32)
    b_ref[...] = jnp.dot(h, wj_ref[...], preferred_element_type=jnp.float32)


EBLK = 6400


def _edge_body(ea_ref, ew1_ref, eb1_ref, wf_ref, bf_ref, ce_ref):
    t = jax.nn.silu(
        jnp.dot(ea_ref[...], ew1_ref[...], preferred_element_type=jnp.float32)
        + eb1_ref[...])
    ce_ref[...] = jnp.dot(t, wf_ref[...], preferred_element_type=jnp.float32) + bf_ref[...]


def _sc_body(a_hbm, b_hbm, ce_hbm, dst_hbm, src_hbm, zeros_hbm, out_hbm,
             acc_sh, dst_v, src_v, buf_a, buf_b, vbuf, sem_a, sem_b):
    c = lax.axis_index("c")
    s = lax.axis_index("s")

    @pl.when(s == 0)
    def _():
        pltpu.sync_copy(zeros_hbm, acc_sh)

    marker = jnp.where(lax.iota(jnp.int32, 16) == 0,
                       jnp.float32(1.0), jnp.float32(0.0))

    def _init_row(i, carry):
        vbuf[i, pl.ds(H, 16)] = marker
        return carry

    lax.fori_loop(0, CHUNK, _init_row, 0)
    plsc.subcore_barrier()

    base0 = c * (E // NC) + s * EP_TILE

    def _chunk(k, carry):
        base = base0 + k * CHUNK
        pltpu.sync_copy(dst_hbm.at[pl.ds(base, CHUNK)], dst_v)
        pltpu.sync_copy(src_hbm.at[pl.ds(base, CHUNK)], src_v)
        cp_a = pltpu.async_copy(a_hbm.at[dst_v], buf_a, sem_a)
        cp_b = pltpu.async_copy(b_hbm.at[src_v], buf_b, sem_b)
        pltpu.sync_copy(ce_hbm.at[pl.ds(base, CHUNK), :], vbuf.at[:, pl.ds(0, H)])
        cp_a.wait()
        cp_b.wait()

        def _row(i, rcarry):
            for j in range(H // 16):
                sl = pl.ds(j * 16, 16)
                t = buf_a[i, sl] + buf_b[i, sl] + vbuf[i, sl]
                vbuf[i, sl] = t / (1.0 + jnp.exp(-t))
            return rcarry

        lax.fori_loop(0, CHUNK, _row, 0)
        pltpu.sync_copy(vbuf, acc_sh.at[dst_v], add=True)
        return carry

    lax.fori_loop(0, NCHUNK, _chunk, 0)
    plsc.subcore_barrier()

    rpt = NPAD // NS
    pltpu.sync_copy(acc_sh.at[pl.ds(s * rpt, rpt), :],
                    out_hbm.at[c, pl.ds(s * rpt, rpt), :])


def _final_body(p0_ref, p1_ref, x_ref, w2_ref, b2_ref, uw1a_ref, uw1b_ref,
                ub1_ref, uw2_ref, ub2_ref, g_ref, beta_ref, o_ref):
    ssum = p0_ref[:, 0:H] + p1_ref[:, 0:H]
    cnt = p0_ref[:, H:H + 1] + p1_ref[:, H:H + 1]
    t = jnp.dot(ssum, w2_ref[...], preferred_element_type=jnp.float32)
    agg = (t + cnt * b2_ref[...]) / jnp.maximum(cnt, 1.0)
    pre = (jnp.dot(x_ref[...], uw1a_ref[...], preferred_element_type=jnp.float32)
           + jnp.dot(agg, uw1b_ref[...], preferred_element_type=jnp.float32)
           + ub1_ref[...])
    upd = jnp.dot(jax.nn.silu(pre), uw2_ref[...],
                  preferred_element_type=jnp.float32) + ub2_ref[...]
    o0 = x_ref[...] + upd
    mu = jnp.mean(o0, axis=-1, keepdims=True)
    var = jnp.mean((o0 - mu) ** 2, axis=-1, keepdims=True)
    o_ref[...] = (o0 - mu) * lax.rsqrt(var + 1e-5) * g_ref[...] + beta_ref[...]


def kernel(x, edge_index, edge_attr, node_w1, node_b1, node_w2, node_b2,
           edge_w1, edge_b1, edge_w2, edge_b2, msg_w1, msg_b1, msg_w2, msg_b2,
           upd_w1, upd_b1, upd_w2, upd_b2, ln_g, ln_b):
    f32 = jnp.float32
    dst = edge_index[1].astype(jnp.int32)
    src = edge_index[0].astype(jnp.int32)

    w_i = msg_w1[:H]
    w_j = msg_w1[H:2 * H]
    w_e = msg_w1[2 * H:]
    wf = jnp.dot(edge_w2, w_e, preferred_element_type=f32)
    bf = (jnp.dot(edge_b2[None, :], w_e, preferred_element_type=f32)
          + msg_b1[None, :])

    a_tab, b_tab = pl.pallas_call(
        _node_body,
        out_shape=(jax.ShapeDtypeStruct((N, H), f32),
                   jax.ShapeDtypeStruct((N, H), f32)),
    )(x, node_w1, node_b1[None, :], node_w2, node_b2[None, :], w_i, w_j)

    ce = pl.pallas_call(
        _edge_body,
        grid=(E // EBLK,),
        in_specs=[
            pl.BlockSpec((EBLK, DE), lambda i: (i, 0)),
            pl.BlockSpec((DE, H), lambda i: (0, 0)),
            pl.BlockSpec((1, H), lambda i: (0, 0)),
            pl.BlockSpec((H, H), lambda i: (0, 0)),
            pl.BlockSpec((1, H), lambda i: (0, 0)),
        ],
        out_specs=pl.BlockSpec((EBLK, H), lambda i: (i, 0)),
        out_shape=jax.ShapeDtypeStruct((E, H), f32),
    )(edge_attr, edge_w1, edge_b1[None, :], wf, bf)

    mesh = plsc.VectorSubcoreMesh(core_axis_name="c", subcore_axis_name="s")
    zeros = jnp.zeros((NPAD, AW), f32)
    sc_fn = pl.kernel(
        _sc_body,
        out_type=jax.ShapeDtypeStruct((NC, NPAD, AW), f32),
        mesh=mesh,
        compiler_params=pltpu.CompilerParams(use_tc_tiling_on_sc=False),
        scratch_types=[
            pltpu.VMEM_SHARED((NPAD, AW), f32),
            pltpu.VMEM((CHUNK,), jnp.int32),
            pltpu.VMEM((CHUNK,), jnp.int32),
            pltpu.VMEM((CHUNK, H), f32),
            pltpu.VMEM((CHUNK, H), f32),
            pltpu.VMEM((CHUNK, AW), f32),
            pltpu.SemaphoreType.DMA,
            pltpu.SemaphoreType.DMA,
        ],
    )
    parts = sc_fn(a_tab, b_tab, ce, dst, src, zeros)

    out = pl.pallas_call(
        _final_body,
        out_shape=jax.ShapeDtypeStruct((N, D), f32),
    )(parts[0][:N], parts[1][:N], x, msg_w2, msg_b2[None, :],
      upd_w1[:D], upd_w1[D:], upd_b1[None, :], upd_w2, upd_b2[None, :],
      ln_g[None, :], ln_b[None, :])
    return out


# speedup vs baseline: 4.2957x; 3.3489x over previous
import functools

import jax
import jax.numpy as jnp
from jax import lax
from jax.experimental import pallas as pl
from jax.experimental.pallas import tpu as pltpu
from jax.experimental.pallas import tpu_sc as plsc

N = 10000
E = 320000
D = 128
DE = 16
H = 128

NC = 2
NS = 16
CHUNK = 40
EP_TILE = E // (NC * NS)
NCHUNK = EP_TILE // CHUNK
AW = H + 16
NPAD = 10240


def _f32(x):
    return x.astype(jnp.float32)


def _node_body(x_ref, nw1_ref, nb1_ref, nw2_ref, nb2_ref, wi_ref, wj_ref,
               a_ref, b_ref):
    h1 = jax.nn.silu(
        jnp.dot(x_ref[...], nw1_ref[...], preferred_element_type=jnp.float32)
        + nb1_ref[...])
    h = jnp.dot(h1, nw2_ref[...], preferred_element_type=jnp.float32) + nb2_ref[...]
    a_ref[...] = jnp.dot(h, wi_ref[...], preferred_element_type=jnp.float32)
    b_ref[...] = jnp.dot(h, wj_ref[...], preferred_element_type=jnp.float32)


EBLK = 6400


def _edge_body(ea_ref, ew1_ref, eb1_ref, wf_ref, bf_ref, ce_ref):
    t = jax.nn.silu(
        jnp.dot(ea_ref[...], ew1_ref[...], preferred_element_type=jnp.float32)
        + eb1_ref[...])
    ce_ref[...] = jnp.dot(t, wf_ref[...], preferred_element_type=jnp.float32) + bf_ref[...]


def _sc_body(a_hbm, b_hbm, ce_hbm, dst_hbm, src_hbm, zeros_hbm, out_hbm,
             acc_sh, dst_v0, dst_v1, src_v0, src_v1, ba0, ba1, bb0, bb1,
             vb0, vb1, sem_a0, sem_a1, sem_b0, sem_b1, sem_c0, sem_c1,
             sem_i0, sem_i1):
    c = lax.axis_index("c")
    s = lax.axis_index("s")
    dst_v = (dst_v0, dst_v1)
    src_v = (src_v0, src_v1)
    buf_a = (ba0, ba1)
    buf_b = (bb0, bb1)
    vbuf = (vb0, vb1)
    sem_a = (sem_a0, sem_a1)
    sem_b = (sem_b0, sem_b1)
    sem_c = (sem_c0, sem_c1)
    sem_i = (sem_i0, sem_i1)

    @pl.when(s == 0)
    def _():
        pltpu.sync_copy(zeros_hbm, acc_sh)

    marker = jnp.where(lax.iota(jnp.int32, 16) == 0,
                       jnp.float32(1.0), jnp.float32(0.0))
    for b in range(2):
        @plsc.parallel_loop(0, CHUNK)
        def _init_row(i, _b=b):
            vbuf[_b][i, pl.ds(H, 16)] = marker
    plsc.subcore_barrier()

    base0 = c * (E // NC) + s * EP_TILE

    def _issue_idx(k, b):
        base = base0 + k * CHUNK
        pltpu.async_copy(dst_hbm.at[pl.ds(base, CHUNK)], dst_v[b], sem_i[b])
        pltpu.async_copy(src_hbm.at[pl.ds(base, CHUNK)], src_v[b], sem_i[b])

    def _wait_idx(b):
        pltpu.make_async_copy(dst_hbm.at[pl.ds(0, CHUNK)], dst_v[b], sem_i[b]).wait()
        pltpu.make_async_copy(src_hbm.at[pl.ds(0, CHUNK)], src_v[b], sem_i[b]).wait()

    def _issue_gathers(b):
        pltpu.async_copy(a_hbm.at[dst_v[b]], buf_a[b], sem_a[b])
        pltpu.async_copy(b_hbm.at[src_v[b]], buf_b[b], sem_b[b])

    def _issue_ce(k, b):
        base = base0 + k * CHUNK
        pltpu.async_copy(ce_hbm.at[pl.ds(base, CHUNK), :],
                         vbuf[b].at[:, pl.ds(0, H)], sem_c[b])

    def _wait_chunk(b):
        pltpu.make_async_copy(a_hbm.at[dst_v[b]], buf_a[b], sem_a[b]).wait()
        pltpu.make_async_copy(b_hbm.at[src_v[b]], buf_b[b], sem_b[b]).wait()
        pltpu.make_async_copy(ce_hbm.at[pl.ds(0, CHUNK), :],
                              vbuf[b].at[:, pl.ds(0, H)], sem_c[b]).wait()

    _issue_idx(0, 0)
    _wait_idx(0)
    _issue_gathers(0)
    _issue_ce(0, 0)
    _issue_idx(1, 1)
    _issue_ce(1, 1)

    def _outer(g, carry):
        for b in range(2):
            k = 2 * g + b
            bp = 1 - b

            @pl.when(k + 1 < NCHUNK)
            def _():
                _wait_idx(bp)
                _issue_gathers(bp)

            _wait_chunk(b)

            @plsc.parallel_loop(0, CHUNK, unroll=2)
            def _row(i, _b=b):
                for j in range(H // 16):
                    sl = pl.ds(j * 16, 16)
                    t = buf_a[_b][i, sl] + buf_b[_b][i, sl] + vbuf[_b][i, sl]
                    vbuf[_b][i, sl] = t / (1.0 + jnp.exp(-t))

            pltpu.sync_copy(vbuf[b], acc_sh.at[dst_v[b]], add=True)

            @pl.when(k + 2 < NCHUNK)
            def _():
                _issue_ce(k + 2, b)
                _issue_idx(k + 2, b)
        return carry

    lax.fori_loop(0, NCHUNK // 2, _outer, 0)
    plsc.subcore_barrier()

    rpt = NPAD // NS
    pltpu.sync_copy(acc_sh.at[pl.ds(s * rpt, rpt), :],
                    out_hbm.at[c, pl.ds(s * rpt, rpt), :])


def _final_body(p0_ref, p1_ref, x_ref, w2_ref, b2_ref, uw1a_ref, uw1b_ref,
                ub1_ref, uw2_ref, ub2_ref, g_ref, beta_ref, o_ref):
    ssum = p0_ref[:, 0:H] + p1_ref[:, 0:H]
    cnt = p0_ref[:, H:H + 1] + p1_ref[:, H:H + 1]
    t = jnp.dot(ssum, w2_ref[...], preferred_element_type=jnp.float32)
    agg = (t + cnt * b2_ref[...]) / jnp.maximum(cnt, 1.0)
    pre = (jnp.dot(x_ref[...], uw1a_ref[...], preferred_element_type=jnp.float32)
           + jnp.dot(agg, uw1b_ref[...], preferred_element_type=jnp.float32)
           + ub1_ref[...])
    upd = jnp.dot(jax.nn.silu(pre), uw2_ref[...],
                  preferred_element_type=jnp.float32) + ub2_ref[...]
    o0 = x_ref[...] + upd
    mu = jnp.mean(o0, axis=-1, keepdims=True)
    var = jnp.mean((o0 - mu) ** 2, axis=-1, keepdims=True)
    o_ref[...] = (o0 - mu) * lax.rsqrt(var + 1e-5) * g_ref[...] + beta_ref[...]


def kernel(x, edge_index, edge_attr, node_w1, node_b1, node_w2, node_b2,
           edge_w1, edge_b1, edge_w2, edge_b2, msg_w1, msg_b1, msg_w2, msg_b2,
           upd_w1, upd_b1, upd_w2, upd_b2, ln_g, ln_b):
    f32 = jnp.float32
    dst = edge_index[1].astype(jnp.int32)
    src = edge_index[0].astype(jnp.int32)

    w_i = msg_w1[:H]
    w_j = msg_w1[H:2 * H]
    w_e = msg_w1[2 * H:]
    wf = jnp.dot(edge_w2, w_e, preferred_element_type=f32)
    bf = (jnp.dot(edge_b2[None, :], w_e, preferred_element_type=f32)
          + msg_b1[None, :])

    a_tab, b_tab = pl.pallas_call(
        _node_body,
        out_shape=(jax.ShapeDtypeStruct((N, H), f32),
                   jax.ShapeDtypeStruct((N, H), f32)),
    )(x, node_w1, node_b1[None, :], node_w2, node_b2[None, :], w_i, w_j)

    ce = pl.pallas_call(
        _edge_body,
        grid=(E // EBLK,),
        in_specs=[
            pl.BlockSpec((EBLK, DE), lambda i: (i, 0)),
            pl.BlockSpec((DE, H), lambda i: (0, 0)),
            pl.BlockSpec((1, H), lambda i: (0, 0)),
            pl.BlockSpec((H, H), lambda i: (0, 0)),
            pl.BlockSpec((1, H), lambda i: (0, 0)),
        ],
        out_specs=pl.BlockSpec((EBLK, H), lambda i: (i, 0)),
        out_shape=jax.ShapeDtypeStruct((E, H), f32),
    )(edge_attr, edge_w1, edge_b1[None, :], wf, bf)

    mesh = plsc.VectorSubcoreMesh(core_axis_name="c", subcore_axis_name="s")
    zeros = jnp.zeros((NPAD, AW), f32)
    sc_fn = pl.kernel(
        _sc_body,
        out_type=jax.ShapeDtypeStruct((NC, NPAD, AW), f32),
        mesh=mesh,
        compiler_params=pltpu.CompilerParams(use_tc_tiling_on_sc=False),
        scratch_types=(
            [pltpu.VMEM_SHARED((NPAD, AW), f32)]
            + [pltpu.VMEM((CHUNK,), jnp.int32)] * 4
            + [pltpu.VMEM((CHUNK, H), f32)] * 4
            + [pltpu.VMEM((CHUNK, AW), f32)] * 2
            + [pltpu.SemaphoreType.DMA] * 8
        ),
    )
    parts = sc_fn(a_tab, b_tab, ce, dst, src, zeros)

    out = pl.pallas_call(
        _final_body,
        out_shape=jax.ShapeDtypeStruct((N, D), f32),
    )(parts[0][:N], parts[1][:N], x, msg_w2, msg_b2[None, :],
      upd_w1[:D], upd_w1[D:], upd_b1[None, :], upd_w2, upd_b2[None, :],
      ln_g[None, :], ln_b[None, :])
    return out


# speedup vs baseline: 4.6434x; 1.0810x over previous
import functools

import jax
import jax.numpy as jnp
from jax import lax
from jax.experimental import pallas as pl
from jax.experimental.pallas import tpu as pltpu
from jax.experimental.pallas import tpu_sc as plsc

N = 10000
E = 320000
D = 128
DE = 16
H = 128

NC = 2
NS = 16
CHUNK = 40
EP_TILE = E // (NC * NS)
NCHUNK = EP_TILE // CHUNK
AW = H + 16
NPAD = 10240


def _f32(x):
    return x.astype(jnp.float32)


def _node_body(x_ref, nw1_ref, nb1_ref, nw2_ref, nb2_ref, wi_ref, wj_ref,
               a_ref, b_ref):
    h1 = jax.nn.silu(
        jnp.dot(x_ref[...], nw1_ref[...], preferred_element_type=jnp.float32)
        + nb1_ref[...])
    h = jnp.dot(h1, nw2_ref[...], preferred_element_type=jnp.float32) + nb2_ref[...]
    a_ref[...] = jnp.dot(h, wi_ref[...], preferred_element_type=jnp.float32)
    b_ref[...] = jnp.dot(h, wj_ref[...], preferred_element_type=jnp.float32)


EBLK = 6400


def _edge_body(eat_ref, ew1_ref, eb1_ref, wf_ref, bf_ref, ce_ref):
    t = jax.nn.silu(
        lax.dot_general(eat_ref[...], ew1_ref[...], (((0,), (0,)), ((), ())),
                        preferred_element_type=jnp.float32)
        + eb1_ref[...])
    ce_ref[...] = jnp.dot(t, wf_ref[...], preferred_element_type=jnp.float32) + bf_ref[...]


def _sc_body(a_hbm, b_hbm, ce_hbm, dst_hbm, src_hbm, zeros_hbm, msg_hbm,
             cnt_hbm, acc_sh, dst_v0, dst_v1, src_v0, src_v1, ba0, ba1, bb0, bb1,
             vb0, vb1, sem_a0, sem_a1, sem_b0, sem_b1, sem_c0, sem_c1,
             sem_i0, sem_i1):
    c = lax.axis_index("c")
    s = lax.axis_index("s")
    dst_v = (dst_v0, dst_v1)
    src_v = (src_v0, src_v1)
    buf_a = (ba0, ba1)
    buf_b = (bb0, bb1)
    vbuf = (vb0, vb1)
    sem_a = (sem_a0, sem_a1)
    sem_b = (sem_b0, sem_b1)
    sem_c = (sem_c0, sem_c1)
    sem_i = (sem_i0, sem_i1)

    @pl.when(s == 0)
    def _():
        pltpu.sync_copy(zeros_hbm, acc_sh)

    marker = jnp.where(lax.iota(jnp.int32, 16) == 0,
                       jnp.float32(1.0), jnp.float32(0.0))
    for b in range(2):
        @plsc.parallel_loop(0, CHUNK)
        def _init_row(i, _b=b):
            vbuf[_b][i, pl.ds(H, 16)] = marker
    plsc.subcore_barrier()

    base0 = c * (E // NC) + s * EP_TILE

    def _issue_idx(k, b):
        base = base0 + k * CHUNK
        pltpu.async_copy(dst_hbm.at[pl.ds(base, CHUNK)], dst_v[b], sem_i[b])
        pltpu.async_copy(src_hbm.at[pl.ds(base, CHUNK)], src_v[b], sem_i[b])

    def _wait_idx(b):
        pltpu.make_async_copy(dst_hbm.at[pl.ds(0, CHUNK)], dst_v[b], sem_i[b]).wait()
        pltpu.make_async_copy(src_hbm.at[pl.ds(0, CHUNK)], src_v[b], sem_i[b]).wait()

    def _issue_gathers(b):
        pltpu.async_copy(a_hbm.at[dst_v[b]], buf_a[b], sem_a[b])
        pltpu.async_copy(b_hbm.at[src_v[b]], buf_b[b], sem_b[b])

    def _issue_ce(k, b):
        base = base0 + k * CHUNK
        pltpu.async_copy(ce_hbm.at[pl.ds(base, CHUNK), :],
                         vbuf[b].at[:, pl.ds(0, H)], sem_c[b])

    def _wait_chunk(b):
        pltpu.make_async_copy(a_hbm.at[dst_v[b]], buf_a[b], sem_a[b]).wait()
        pltpu.make_async_copy(b_hbm.at[src_v[b]], buf_b[b], sem_b[b]).wait()
        pltpu.make_async_copy(ce_hbm.at[pl.ds(0, CHUNK), :],
                              vbuf[b].at[:, pl.ds(0, H)], sem_c[b]).wait()

    _issue_idx(0, 0)
    _wait_idx(0)
    _issue_gathers(0)
    _issue_ce(0, 0)
    _issue_idx(1, 1)
    _issue_ce(1, 1)

    def _outer(g, carry):
        for b in range(2):
            k = 2 * g + b
            bp = 1 - b

            @pl.when(k + 1 < NCHUNK)
            def _():
                _wait_idx(bp)
                _issue_gathers(bp)

            _wait_chunk(b)

            @plsc.parallel_loop(0, CHUNK, unroll=2)
            def _row(i, _b=b):
                for j in range(H // 16):
                    sl = pl.ds(j * 16, 16)
                    t = buf_a[_b][i, sl] + buf_b[_b][i, sl] + vbuf[_b][i, sl]
                    vbuf[_b][i, sl] = t / (1.0 + jnp.exp(-t))

            pltpu.sync_copy(vbuf[b], acc_sh.at[dst_v[b]], add=True)

            @pl.when(k + 2 < NCHUNK)
            def _():
                _issue_ce(k + 2, b)
                _issue_idx(k + 2, b)
        return carry

    lax.fori_loop(0, NCHUNK // 2, _outer, 0)
    plsc.subcore_barrier()

    rpt = NPAD // NS
    row = pl.ds(s * rpt, rpt)
    pltpu.sync_copy(acc_sh.at[row, pl.ds(0, H)], msg_hbm.at[c, row, :])
    pltpu.sync_copy(acc_sh.at[row, pl.ds(H, 16)], cnt_hbm.at[c, row, :])


def _final_body(msg_ref, cntp_ref, x_ref, w2_ref, b2_ref, uw1a_ref, uw1b_ref,
                ub1_ref, uw2_ref, ub2_ref, g_ref, beta_ref, o_ref):
    ssum = msg_ref[0, 0:N, :] + msg_ref[1, 0:N, :]
    cnt = cntp_ref[0, 0:N, 0:1] + cntp_ref[1, 0:N, 0:1]
    t = jnp.dot(ssum, w2_ref[...], preferred_element_type=jnp.float32)
    agg = (t + cnt * b2_ref[...]) / jnp.maximum(cnt, 1.0)
    pre = (jnp.dot(x_ref[...], uw1a_ref[...], preferred_element_type=jnp.float32)
           + jnp.dot(agg, uw1b_ref[...], preferred_element_type=jnp.float32)
           + ub1_ref[...])
    upd = jnp.dot(jax.nn.silu(pre), uw2_ref[...],
                  preferred_element_type=jnp.float32) + ub2_ref[...]
    o0 = x_ref[...] + upd
    mu = jnp.mean(o0, axis=-1, keepdims=True)
    var = jnp.mean((o0 - mu) ** 2, axis=-1, keepdims=True)
    o_ref[...] = (o0 - mu) * lax.rsqrt(var + 1e-5) * g_ref[...] + beta_ref[...]


def kernel(x, edge_index, edge_attr, node_w1, node_b1, node_w2, node_b2,
           edge_w1, edge_b1, edge_w2, edge_b2, msg_w1, msg_b1, msg_w2, msg_b2,
           upd_w1, upd_b1, upd_w2, upd_b2, ln_g, ln_b):
    f32 = jnp.float32
    dst = edge_index[1].astype(jnp.int32)
    src = edge_index[0].astype(jnp.int32)

    w_i = msg_w1[:H]
    w_j = msg_w1[H:2 * H]
    w_e = msg_w1[2 * H:]
    wf = jnp.dot(edge_w2, w_e, preferred_element_type=f32)
    bf = (jnp.dot(edge_b2[None, :], w_e, preferred_element_type=f32)
          + msg_b1[None, :])

    a_tab, b_tab = pl.pallas_call(
        _node_body,
        out_shape=(jax.ShapeDtypeStruct((N, H), f32),
                   jax.ShapeDtypeStruct((N, H), f32)),
    )(x, node_w1, node_b1[None, :], node_w2, node_b2[None, :], w_i, w_j)

    ce = pl.pallas_call(
        _edge_body,
        grid=(E // EBLK,),
        in_specs=[
            pl.BlockSpec((DE, EBLK), lambda i: (0, i)),
            pl.BlockSpec((DE, H), lambda i: (0, 0)),
            pl.BlockSpec((1, H), lambda i: (0, 0)),
            pl.BlockSpec((H, H), lambda i: (0, 0)),
            pl.BlockSpec((1, H), lambda i: (0, 0)),
        ],
        out_specs=pl.BlockSpec((EBLK, H), lambda i: (i, 0)),
        out_shape=jax.ShapeDtypeStruct((E, H), f32),
    )(edge_attr.T, edge_w1, edge_b1[None, :], wf, bf)

    mesh = plsc.VectorSubcoreMesh(core_axis_name="c", subcore_axis_name="s")
    zeros = jnp.zeros((NPAD, AW), f32)
    sc_fn = pl.kernel(
        _sc_body,
        out_type=(jax.ShapeDtypeStruct((NC, NPAD, H), f32),
                  jax.ShapeDtypeStruct((NC, NPAD, 16), f32)),
        mesh=mesh,
        compiler_params=pltpu.CompilerParams(use_tc_tiling_on_sc=False),
        scratch_types=(
            [pltpu.VMEM_SHARED((NPAD, AW), f32)]
            + [pltpu.VMEM((CHUNK,), jnp.int32)] * 4
            + [pltpu.VMEM((CHUNK, H), f32)] * 4
            + [pltpu.VMEM((CHUNK, AW), f32)] * 2
            + [pltpu.SemaphoreType.DMA] * 8
        ),
    )
    msg_parts, cnt_parts = sc_fn(a_tab, b_tab, ce, dst, src, zeros)

    out = pl.pallas_call(
        _final_body,
        out_shape=jax.ShapeDtypeStruct((N, D), f32),
    )(msg_parts, cnt_parts, x, msg_w2, msg_b2[None, :],
      upd_w1[:D], upd_w1[D:], upd_b1[None, :], upd_w2, upd_b2[None, :],
      ln_g[None, :], ln_b[None, :])
    return out
